# baseline (speedup 1.0000x reference)
import functools

import jax
import jax.numpy as jnp
from jax import lax
from jax.experimental import pallas as pl
from jax.experimental.pallas import tpu as pltpu
from jax.experimental.pallas import tpu_sc as plsc

USER_NUM = 1000000
ITEM_NUM = 1000000
FACTOR = 16
BATCH = 16384

L = 16
NW = 32
BPW = BATCH // NW
NCHUNK = 4
CHUNK = BPW // NCHUNK
NBLK = BPW // L


def _body(user_hbm, item_i_hbm, item_j_hbm, eu_hbm, ei_hbm, bias_hbm,
          out_pi, out_pj, out_pw,
          idx_u, idx_i, idx_j,
          rows_u, rows_i, rows_j,
          bias_i_v, bias_j_v,
          pw_v, pred_i_v, pred_j_v, sem):
    wid = lax.axis_index("s") * 2 + lax.axis_index("c")
    base = wid * BPW

    pltpu.sync_copy(user_hbm.at[wid], idx_u)
    pltpu.sync_copy(item_i_hbm.at[wid], idx_i)
    pltpu.sync_copy(item_j_hbm.at[wid], idx_j)

    copies = []
    for c in range(NCHUNK):
        dst = pl.ds(c * CHUNK, CHUNK)
        copies.append(pltpu.async_copy(eu_hbm.at[idx_u.at[c]], rows_u.at[dst], sem))
        copies.append(pltpu.async_copy(ei_hbm.at[idx_i.at[c]], rows_i.at[dst], sem))
        copies.append(pltpu.async_copy(ei_hbm.at[idx_j.at[c]], rows_j.at[dst], sem))
        copies.append(pltpu.async_copy(bias_hbm.at[idx_i.at[c]], bias_i_v.at[dst], sem))
        copies.append(pltpu.async_copy(bias_hbm.at[idx_j.at[c]], bias_j_v.at[dst], sem))
    for cp in copies:
        cp.wait()

    zeros = jnp.zeros((L,), jnp.int32)
    iota = lax.iota(jnp.int32, L)

    def block(b, carry):
        rbase = b * L
        ridx = rbase + iota
        acc_i = bias_i_v[pl.ds(rbase, L)]
        acc_j = bias_j_v[pl.ds(rbase, L)]
        for f in range(FACTOR):
            cf = jnp.full((L,), f, jnp.int32)
            uc = plsc.load_gather(rows_u, [ridx, cf])
            ic = plsc.load_gather(rows_i, [ridx, cf])
            jc = plsc.load_gather(rows_j, [ridx, cf])
            pwc = uc * ic
            plsc.store_scatter(pw_v, [ridx, cf], pwc)
            acc_i = acc_i + pwc
            acc_j = acc_j + uc * jc
        pred_i_v[pl.ds(rbase, L)] = acc_i
        pred_j_v[pl.ds(rbase, L)] = acc_j
        return carry

    lax.fori_loop(0, NBLK, block, 0)

    pltpu.sync_copy(pred_i_v, out_pi.at[wid])
    pltpu.sync_copy(pred_j_v, out_pj.at[wid])
    pltpu.sync_copy(pw_v, out_pw.at[pl.ds(base, BPW)])


@jax.jit
def _bpr_sc(user, item_i, item_j, embed_user, embed_item, item_biases):
    mesh = plsc.VectorSubcoreMesh(core_axis_name="c", subcore_axis_name="s")
    kern = functools.partial(
        pl.kernel,
        mesh=mesh,
        compiler_params=pltpu.CompilerParams(
            needs_layout_passes=False, use_tc_tiling_on_sc=False),
        out_type=[
            jax.ShapeDtypeStruct((NW, BPW), jnp.float32),
            jax.ShapeDtypeStruct((NW, BPW), jnp.float32),
            jax.ShapeDtypeStruct((BATCH, FACTOR), jnp.float32),
        ],
        scratch_types=[
            pltpu.VMEM((NCHUNK, CHUNK), jnp.int32),
            pltpu.VMEM((NCHUNK, CHUNK), jnp.int32),
            pltpu.VMEM((NCHUNK, CHUNK), jnp.int32),
            pltpu.VMEM((BPW, FACTOR), jnp.float32),
            pltpu.VMEM((BPW, FACTOR), jnp.float32),
            pltpu.VMEM((BPW, FACTOR), jnp.float32),
            pltpu.VMEM((BPW,), jnp.float32),
            pltpu.VMEM((BPW,), jnp.float32),
            pltpu.VMEM((BPW, FACTOR), jnp.float32),
            pltpu.VMEM((BPW,), jnp.float32),
            pltpu.VMEM((BPW,), jnp.float32),
            pltpu.SemaphoreType.DMA,
        ],
    )(_body)
    pi, pj, pw = kern(user, item_i, item_j, embed_user, embed_item, item_biases)
    return pi, pj, pw


def kernel(user, item_i, item_j, embed_user, embed_item, item_biases):
    u = user.astype(jnp.int32).reshape(NW, NCHUNK, CHUNK)
    ii = item_i.astype(jnp.int32).reshape(NW, NCHUNK, CHUNK)
    ij = item_j.astype(jnp.int32).reshape(NW, NCHUNK, CHUNK)
    pi, pj, pw = _bpr_sc(u, ii, ij, embed_user, embed_item,
                         item_biases.reshape(ITEM_NUM))
    return pi.reshape(BATCH), pj.reshape(BATCH), pw
